# baseline (speedup 1.0000x reference)
import jax
import jax.numpy as jnp
from jax import lax
from jax.experimental import pallas as pl
from jax.experimental.pallas import tpu as pltpu, tpu_sc as plsc

NC = 2
NS = 16
NW = NC * NS

VOCAB = 1000000
DIM = 64
ROW = 128
CHUNK = 256
NBLK = VOCAB // ROW
TAIL = VOCAB - NBLK * ROW


def _transpose_body(tabT_hbm, tail_hbm, tlin_hbm, blk_v, trn_v):
    wid = lax.axis_index("s") * NC + lax.axis_index("c")
    lo = wid * NBLK // NW
    hi = (wid + 1) * NBLK // NW

    iotas = [lax.iota(jnp.int32, 16) + 16 * j for j in range(4)]

    @pl.loop(lo, hi)
    def _(t):
        c0 = pl.multiple_of(t * ROW, ROW)
        pltpu.sync_copy(tabT_hbm.at[:, pl.ds(c0, ROW)], blk_v)

        @pl.loop(0, ROW)
        def _(r):
            rvec = jnp.full((16,), r, jnp.int32)
            for j in range(4):
                v = plsc.load_gather(blk_v, [iotas[j], rvec])
                trn_v[r, pl.ds(16 * j, 16)] = v

        pltpu.sync_copy(trn_v, tlin_hbm.at[pl.ds(c0, ROW)])

    @pl.when(wid == NW - 1)
    def _():
        pltpu.sync_copy(tail_hbm, trn_v.at[pl.ds(0, TAIL)])
        pltpu.sync_copy(trn_v.at[pl.ds(0, TAIL)],
                        tlin_hbm.at[pl.ds(NBLK * ROW, TAIL)])


def _gather_body(idx_hbm, tlin_hbm, out_hbm, idx_v0, idx_v1, rows_v0,
                 rows_v1, si, sg, so):
    wid = lax.axis_index("s") * NC + lax.axis_index("c")
    tot = idx_hbm.shape[0]
    per_w = tot // NW
    n = per_w // CHUNK
    base = wid * per_w
    idx_v = [idx_v0, idx_v1]
    rows_v = [rows_v0, rows_v1]

    def idx_start(j, b):
        pltpu.async_copy(
            idx_hbm.at[pl.ds(base + j * CHUNK, CHUNK)], idx_v[b], si.at[b])

    def idx_wait(b):
        pltpu.make_async_copy(
            idx_hbm.at[pl.ds(0, CHUNK)], idx_v[b], si.at[b]).wait()

    def gather_start(b):
        pltpu.async_copy(tlin_hbm.at[idx_v[b]], rows_v[b], sg.at[b])

    def gather_wait(b):
        pltpu.make_async_copy(
            tlin_hbm.at[idx_v[b]], rows_v[b], sg.at[b]).wait()

    def out_start(j, b):
        pltpu.async_copy(
            rows_v[b], out_hbm.at[pl.ds(base + j * CHUNK, CHUNK)], so.at[b])

    def out_wait(b):
        pltpu.make_async_copy(
            rows_v[b], out_hbm.at[pl.ds(0, CHUNK)], so.at[b]).wait()

    idx_start(0, 0)
    idx_start(1, 1)
    idx_wait(0)
    gather_start(0)

    @pl.loop(0, n, step=2)
    def _(j0):
        for t in range(2):
            j = j0 + t
            b = t
            nb = 1 - t
            gather_wait(b)
            out_start(j, b)
            @pl.when(j + 2 < n)
            def _():
                idx_start(j + 2, b)
            @pl.when(j + 1 < n)
            def _():
                idx_wait(nb)
                @pl.when(j >= 1)
                def _():
                    out_wait(nb)
                gather_start(nb)

    out_wait(0)
    out_wait(1)


def _transpose_table(tabT, tail):
    mesh = plsc.VectorSubcoreMesh(core_axis_name="c", subcore_axis_name="s")
    run = pl.kernel(
        _transpose_body,
        out_type=jax.ShapeDtypeStruct((VOCAB, ROW), jnp.float32),
        mesh=mesh,
        scratch_types=[
            pltpu.VMEM((DIM, ROW), jnp.float32),
            pltpu.VMEM((ROW, ROW), jnp.float32),
        ],
        compiler_params=pltpu.CompilerParams(
            use_tc_tiling_on_sc=True, needs_layout_passes=False),
    )
    return run(tabT, tail)


def _embedding_gather(flat_idx, tlin):
    tot = flat_idx.shape[0]
    mesh = plsc.VectorSubcoreMesh(core_axis_name="c", subcore_axis_name="s")
    run = pl.kernel(
        _gather_body,
        out_type=jax.ShapeDtypeStruct((tot, ROW), jnp.float32),
        mesh=mesh,
        scratch_types=[
            pltpu.VMEM((CHUNK,), jnp.int32),
            pltpu.VMEM((CHUNK,), jnp.int32),
            pltpu.VMEM((CHUNK, ROW), jnp.float32),
            pltpu.VMEM((CHUNK, ROW), jnp.float32),
            pltpu.SemaphoreType.DMA((2,)),
            pltpu.SemaphoreType.DMA((2,)),
            pltpu.SemaphoreType.DMA((2,)),
        ],
        compiler_params=pltpu.CompilerParams(use_tc_tiling_on_sc=True),
    )
    return run(flat_idx, tlin)


def kernel(x, table):
    b, l = x.shape
    tail = jnp.pad(table[NBLK * ROW:], ((0, 0), (0, ROW - DIM)))
    tlin = _transpose_table(table.T, tail)
    outp = _embedding_gather(x.reshape(-1), tlin)
    return outp[:, :DIM].reshape(b, l, DIM)


# speedup vs baseline: 1.2216x; 1.2216x over previous
import jax
import jax.numpy as jnp
from jax import lax
from jax.experimental import pallas as pl
from jax.experimental.pallas import tpu as pltpu, tpu_sc as plsc

NC = 2
NS = 16
NW = NC * NS

VOCAB = 1000000
DIM = 64
ROW = 128
CHUNK = 256
NBLK = VOCAB // ROW
TAIL = VOCAB - NBLK * ROW


def _transpose_body(tabT_hbm, tail_hbm, tlin_hbm, blk_v0, blk_v1, trn_v0,
                    trn_v1, si, so):
    wid = lax.axis_index("s") * NC + lax.axis_index("c")
    lo = wid * NBLK // NW
    hi = (wid + 1) * NBLK // NW
    blk_v = [blk_v0, blk_v1]
    trn_v = [trn_v0, trn_v1]

    iotas = [lax.iota(jnp.int32, 16) + 16 * j for j in range(4)]

    def in_start(t, b):
        c0 = pl.multiple_of(t * ROW, ROW)
        pltpu.async_copy(tabT_hbm.at[:, pl.ds(c0, ROW)], blk_v[b], si.at[b])

    def in_wait(b):
        pltpu.make_async_copy(
            tabT_hbm.at[:, pl.ds(0, ROW)], blk_v[b], si.at[b]).wait()

    def out_start(t, b):
        c0 = pl.multiple_of(t * ROW, ROW)
        pltpu.async_copy(trn_v[b], tlin_hbm.at[pl.ds(c0, ROW)], so.at[b])

    def out_wait(b):
        pltpu.make_async_copy(
            trn_v[b], tlin_hbm.at[pl.ds(0, ROW)], so.at[b]).wait()

    def transpose_blk(b):
        @pl.loop(0, ROW, unroll=4)
        def _(r):
            rvec = jnp.full((16,), r, jnp.int32)
            for j in range(4):
                v = plsc.load_gather(blk_v[b], [iotas[j], rvec])
                trn_v[b][r, pl.ds(16 * j, 16)] = v

    in_start(lo, 0)
    in_start(lo + 1, 1)

    @pl.loop(lo, hi)
    def _(t):
        for b in range(2):
            @pl.when((t - lo) % 2 == b)
            def _():
                in_wait(b)
                @pl.when(t - lo >= 2)
                def _():
                    out_wait(b)
                transpose_blk(b)
                out_start(t, b)
                @pl.when(t + 2 < hi)
                def _():
                    in_start(t + 2, b)

    out_wait(0)
    out_wait(1)

    @pl.when(wid == NW - 1)
    def _():
        pltpu.sync_copy(tail_hbm, trn_v0.at[pl.ds(0, TAIL)])
        pltpu.sync_copy(trn_v0.at[pl.ds(0, TAIL)],
                        tlin_hbm.at[pl.ds(NBLK * ROW, TAIL)])


def _gather_body(idx_hbm, tlin_hbm, out_hbm, idx_v0, idx_v1, rows_v0,
                 rows_v1, si, sg, so):
    wid = lax.axis_index("s") * NC + lax.axis_index("c")
    tot = idx_hbm.shape[0]
    per_w = tot // NW
    n = per_w // CHUNK
    base = wid * per_w
    idx_v = [idx_v0, idx_v1]
    rows_v = [rows_v0, rows_v1]

    def idx_start(j, b):
        pltpu.async_copy(
            idx_hbm.at[pl.ds(base + j * CHUNK, CHUNK)], idx_v[b], si.at[b])

    def idx_wait(b):
        pltpu.make_async_copy(
            idx_hbm.at[pl.ds(0, CHUNK)], idx_v[b], si.at[b]).wait()

    def gather_start(b):
        pltpu.async_copy(tlin_hbm.at[idx_v[b]], rows_v[b], sg.at[b])

    def gather_wait(b):
        pltpu.make_async_copy(
            tlin_hbm.at[idx_v[b]], rows_v[b], sg.at[b]).wait()

    def out_start(j, b):
        pltpu.async_copy(
            rows_v[b], out_hbm.at[pl.ds(base + j * CHUNK, CHUNK)], so.at[b])

    def out_wait(b):
        pltpu.make_async_copy(
            rows_v[b], out_hbm.at[pl.ds(0, CHUNK)], so.at[b]).wait()

    idx_start(0, 0)
    idx_start(1, 1)
    idx_wait(0)
    gather_start(0)

    @pl.loop(0, n, step=2)
    def _(j0):
        for t in range(2):
            j = j0 + t
            b = t
            nb = 1 - t
            gather_wait(b)
            out_start(j, b)
            @pl.when(j + 2 < n)
            def _():
                idx_start(j + 2, b)
            @pl.when(j + 1 < n)
            def _():
                idx_wait(nb)
                @pl.when(j >= 1)
                def _():
                    out_wait(nb)
                gather_start(nb)

    out_wait(0)
    out_wait(1)


def _transpose_table(tabT, tail):
    mesh = plsc.VectorSubcoreMesh(core_axis_name="c", subcore_axis_name="s")
    run = pl.kernel(
        _transpose_body,
        out_type=jax.ShapeDtypeStruct((VOCAB, ROW), jnp.float32),
        mesh=mesh,
        scratch_types=[
            pltpu.VMEM((DIM, ROW), jnp.float32),
            pltpu.VMEM((DIM, ROW), jnp.float32),
            pltpu.VMEM((ROW, ROW), jnp.float32),
            pltpu.VMEM((ROW, ROW), jnp.float32),
            pltpu.SemaphoreType.DMA((2,)),
            pltpu.SemaphoreType.DMA((2,)),
        ],
        compiler_params=pltpu.CompilerParams(
            use_tc_tiling_on_sc=True, needs_layout_passes=False),
    )
    return run(tabT, tail)


def _embedding_gather(flat_idx, tlin):
    tot = flat_idx.shape[0]
    mesh = plsc.VectorSubcoreMesh(core_axis_name="c", subcore_axis_name="s")
    run = pl.kernel(
        _gather_body,
        out_type=jax.ShapeDtypeStruct((tot, ROW), jnp.float32),
        mesh=mesh,
        scratch_types=[
            pltpu.VMEM((CHUNK,), jnp.int32),
            pltpu.VMEM((CHUNK,), jnp.int32),
            pltpu.VMEM((CHUNK, ROW), jnp.float32),
            pltpu.VMEM((CHUNK, ROW), jnp.float32),
            pltpu.SemaphoreType.DMA((2,)),
            pltpu.SemaphoreType.DMA((2,)),
            pltpu.SemaphoreType.DMA((2,)),
        ],
        compiler_params=pltpu.CompilerParams(use_tc_tiling_on_sc=True),
    )
    return run(flat_idx, tlin)


def kernel(x, table):
    b, l = x.shape
    tail = jnp.pad(table[NBLK * ROW:], ((0, 0), (0, ROW - DIM)))
    tlin = _transpose_table(table.T, tail)
    outp = _embedding_gather(x.reshape(-1), tlin)
    return outp[:, :DIM].reshape(b, l, DIM)


# speedup vs baseline: 1.4255x; 1.1669x over previous
import jax
import jax.numpy as jnp
from jax import lax
from jax.experimental import pallas as pl
from jax.experimental.pallas import tpu as pltpu, tpu_sc as plsc

NC = 2
NS = 16
NW = NC * NS

VOCAB = 1000000
DIM = 64
ROW = 128
CHUNK = 256
NBLK = VOCAB // ROW
TAIL = VOCAB - NBLK * ROW


def _transpose_body(tabT_hbm, tail_hbm, tlin_hbm, blk_v0, blk_v1, trn_v0,
                    trn_v1, si, so):
    wid = lax.axis_index("s") * NC + lax.axis_index("c")
    lo = wid * NBLK // NW
    hi = (wid + 1) * NBLK // NW
    blk_v = [blk_v0, blk_v1]
    trn_v = [trn_v0, trn_v1]

    riotas = [lax.iota(jnp.int32, 16) + 16 * g for g in range(8)]

    def in_start(t, b):
        c0 = pl.multiple_of(t * ROW, ROW)
        pltpu.async_copy(tabT_hbm.at[:, pl.ds(c0, ROW)], blk_v[b], si.at[b])

    def in_wait(b):
        pltpu.make_async_copy(
            tabT_hbm.at[:, pl.ds(0, ROW)], blk_v[b], si.at[b]).wait()

    def out_start(t, b):
        c0 = pl.multiple_of(t * ROW, ROW)
        pltpu.async_copy(trn_v[b], tlin_hbm.at[pl.ds(c0, ROW)], so.at[b])

    def out_wait(b):
        pltpu.make_async_copy(
            trn_v[b], tlin_hbm.at[pl.ds(0, ROW)], so.at[b]).wait()

    def transpose_blk(b):
        @pl.loop(0, DIM, unroll=2)
        def _(c):
            cvec = jnp.full((16,), c, jnp.int32)
            for g in range(8):
                v = blk_v[b][c, pl.ds(16 * g, 16)]
                plsc.store_scatter(trn_v[b], [riotas[g], cvec], v)

    in_start(lo, 0)
    in_start(lo + 1, 1)

    @pl.loop(lo, hi)
    def _(t):
        for b in range(2):
            @pl.when((t - lo) % 2 == b)
            def _():
                in_wait(b)
                @pl.when(t - lo >= 2)
                def _():
                    out_wait(b)
                transpose_blk(b)
                out_start(t, b)
                @pl.when(t + 2 < hi)
                def _():
                    in_start(t + 2, b)

    out_wait(0)
    out_wait(1)

    @pl.when(wid == NW - 1)
    def _():
        pltpu.sync_copy(tail_hbm, trn_v0.at[pl.ds(0, TAIL)])
        pltpu.sync_copy(trn_v0.at[pl.ds(0, TAIL)],
                        tlin_hbm.at[pl.ds(NBLK * ROW, TAIL)])


def _gather_body(idx_hbm, tlin_hbm, out_hbm, idx_v0, idx_v1, rows_v0,
                 rows_v1, si, sg, so):
    wid = lax.axis_index("s") * NC + lax.axis_index("c")
    tot = idx_hbm.shape[0]
    per_w = tot // NW
    n = per_w // CHUNK
    base = wid * per_w
    idx_v = [idx_v0, idx_v1]
    rows_v = [rows_v0, rows_v1]

    def idx_start(j, b):
        pltpu.async_copy(
            idx_hbm.at[pl.ds(base + j * CHUNK, CHUNK)], idx_v[b], si.at[b])

    def idx_wait(b):
        pltpu.make_async_copy(
            idx_hbm.at[pl.ds(0, CHUNK)], idx_v[b], si.at[b]).wait()

    def gather_start(b):
        pltpu.async_copy(tlin_hbm.at[idx_v[b]], rows_v[b], sg.at[b])

    def gather_wait(b):
        pltpu.make_async_copy(
            tlin_hbm.at[idx_v[b]], rows_v[b], sg.at[b]).wait()

    def out_start(j, b):
        pltpu.async_copy(
            rows_v[b], out_hbm.at[pl.ds(base + j * CHUNK, CHUNK)], so.at[b])

    def out_wait(b):
        pltpu.make_async_copy(
            rows_v[b], out_hbm.at[pl.ds(0, CHUNK)], so.at[b]).wait()

    idx_start(0, 0)
    idx_start(1, 1)
    idx_wait(0)
    gather_start(0)

    @pl.loop(0, n, step=2)
    def _(j0):
        for t in range(2):
            j = j0 + t
            b = t
            nb = 1 - t
            gather_wait(b)
            out_start(j, b)
            @pl.when(j + 2 < n)
            def _():
                idx_start(j + 2, b)
            @pl.when(j + 1 < n)
            def _():
                idx_wait(nb)
                @pl.when(j >= 1)
                def _():
                    out_wait(nb)
                gather_start(nb)

    out_wait(0)
    out_wait(1)


def _transpose_table(tabT, tail):
    mesh = plsc.VectorSubcoreMesh(core_axis_name="c", subcore_axis_name="s")
    run = pl.kernel(
        _transpose_body,
        out_type=jax.ShapeDtypeStruct((VOCAB, ROW), jnp.float32),
        mesh=mesh,
        scratch_types=[
            pltpu.VMEM((DIM, ROW), jnp.float32),
            pltpu.VMEM((DIM, ROW), jnp.float32),
            pltpu.VMEM((ROW, ROW), jnp.float32),
            pltpu.VMEM((ROW, ROW), jnp.float32),
            pltpu.SemaphoreType.DMA((2,)),
            pltpu.SemaphoreType.DMA((2,)),
        ],
        compiler_params=pltpu.CompilerParams(
            use_tc_tiling_on_sc=True, needs_layout_passes=False),
    )
    return run(tabT, tail)


def _embedding_gather(flat_idx, tlin):
    tot = flat_idx.shape[0]
    mesh = plsc.VectorSubcoreMesh(core_axis_name="c", subcore_axis_name="s")
    run = pl.kernel(
        _gather_body,
        out_type=jax.ShapeDtypeStruct((tot, ROW), jnp.float32),
        mesh=mesh,
        scratch_types=[
            pltpu.VMEM((CHUNK,), jnp.int32),
            pltpu.VMEM((CHUNK,), jnp.int32),
            pltpu.VMEM((CHUNK, ROW), jnp.float32),
            pltpu.VMEM((CHUNK, ROW), jnp.float32),
            pltpu.SemaphoreType.DMA((2,)),
            pltpu.SemaphoreType.DMA((2,)),
            pltpu.SemaphoreType.DMA((2,)),
        ],
        compiler_params=pltpu.CompilerParams(use_tc_tiling_on_sc=True),
    )
    return run(flat_idx, tlin)


def kernel(x, table):
    b, l = x.shape
    tail = jnp.pad(table[NBLK * ROW:], ((0, 0), (0, ROW - DIM)))
    tlin = _transpose_table(table.T, tail)
    outp = _embedding_gather(x.reshape(-1), tlin)
    return outp[:, :DIM].reshape(b, l, DIM)


# speedup vs baseline: 2.6100x; 1.8310x over previous
import jax
import jax.numpy as jnp
from jax import lax
from jax.experimental import pallas as pl
from jax.experimental.pallas import tpu as pltpu, tpu_sc as plsc

NC = 2
NS = 16
NW = NC * NS

DIM = 64
ROW = 128
CHUNK = 512


def _gather_body(idx_hbm, table_hbm, out_hbm, idx_v0, idx_v1, rows_v0,
                 rows_v1, si, sg, so):
    wid = lax.axis_index("s") * NC + lax.axis_index("c")
    tot = idx_hbm.shape[0]
    per_w = tot // NW
    n = per_w // CHUNK
    base = wid * per_w
    idx_v = [idx_v0, idx_v1]
    rows_v = [rows_v0, rows_v1]

    def idx_start(j, b):
        pltpu.async_copy(
            idx_hbm.at[pl.ds(base + j * CHUNK, CHUNK)], idx_v[b], si.at[b])

    def idx_wait(b):
        pltpu.make_async_copy(
            idx_hbm.at[pl.ds(0, CHUNK)], idx_v[b], si.at[b]).wait()

    def gather_start(b):
        pltpu.async_copy(table_hbm.at[idx_v[b]], rows_v[b], sg.at[b])

    def gather_wait(b):
        pltpu.make_async_copy(
            table_hbm.at[idx_v[b]], rows_v[b], sg.at[b]).wait()

    def out_start(j, b):
        pltpu.async_copy(
            rows_v[b],
            out_hbm.at[pl.ds(base + j * CHUNK, CHUNK), pl.ds(0, DIM)],
            so.at[b])

    def out_wait(b):
        pltpu.make_async_copy(
            rows_v[b], out_hbm.at[pl.ds(0, CHUNK), pl.ds(0, DIM)],
            so.at[b]).wait()

    idx_start(0, 0)
    idx_start(1, 1)
    idx_wait(0)
    gather_start(0)

    @pl.loop(0, n, step=2)
    def _(j0):
        for t in range(2):
            j = j0 + t
            b = t
            nb = 1 - t
            gather_wait(b)
            out_start(j, b)
            @pl.when(j + 2 < n)
            def _():
                idx_start(j + 2, b)
            @pl.when(j + 1 < n)
            def _():
                idx_wait(nb)
                @pl.when(j >= 1)
                def _():
                    out_wait(nb)
                gather_start(nb)

    out_wait(0)
    out_wait(1)


def _embedding_gather(flat_idx, table):
    tot = flat_idx.shape[0]
    mesh = plsc.VectorSubcoreMesh(core_axis_name="c", subcore_axis_name="s")
    run = pl.kernel(
        _gather_body,
        out_type=jax.ShapeDtypeStruct((tot, ROW), jnp.float32),
        mesh=mesh,
        scratch_types=[
            pltpu.VMEM((CHUNK,), jnp.int32),
            pltpu.VMEM((CHUNK,), jnp.int32),
            pltpu.VMEM((CHUNK, DIM), jnp.float32),
            pltpu.VMEM((CHUNK, DIM), jnp.float32),
            pltpu.SemaphoreType.DMA((2,)),
            pltpu.SemaphoreType.DMA((2,)),
            pltpu.SemaphoreType.DMA((2,)),
        ],
        compiler_params=pltpu.CompilerParams(use_tc_tiling_on_sc=False),
    )
    return run(flat_idx, table)


def kernel(x, table):
    b, l = x.shape
    outp = _embedding_gather(x.reshape(-1), table)
    return outp[:, :DIM].reshape(b, l, DIM)
